# Initial kernel scaffold; baseline (speedup 1.0000x reference)
import jax
import jax.numpy as jnp
from jax.experimental import pallas as pl


def kernel(inputs, word_embeddings):
    raise NotImplementedError("write your pallas kernel here")


# speedup vs baseline: 1.0222x; 1.0222x over previous
import functools

import jax
import jax.numpy as jnp
from jax import lax
from jax.experimental import pallas as pl
from jax.experimental.pallas import tpu as pltpu
from jax.experimental.pallas import tpu_sc as plsc

_NC = 2
_NS = 16
_NW = _NC * _NS

_CHUNK = 128


@functools.lru_cache(maxsize=None)
def _make_lookup(V, D, N):
    assert N % (_NW * _CHUNK) == 0
    b_per_w = N // _NW
    n_chunks = b_per_w // _CHUNK

    mesh = plsc.VectorSubcoreMesh(
        core_axis_name="c", subcore_axis_name="s",
        num_cores=_NC, num_subcores=_NS)

    @functools.partial(
        pl.kernel,
        out_type=jax.ShapeDtypeStruct((N, D), jnp.float32),
        mesh=mesh,
        scratch_types=[
            pltpu.VMEM((n_chunks, _CHUNK), jnp.int32),
            pltpu.VMEM((_CHUNK, D), jnp.float32),
            pltpu.SemaphoreType.DMA,
        ],
        compiler_params=pltpu.CompilerParams(use_tc_tiling_on_sc=False),
    )
    def lookup(idx_hbm, table_hbm, out_hbm, idx_v, rows_v, sem):
        wid = lax.axis_index("s") * _NC + lax.axis_index("c")
        base = wid * b_per_w
        pltpu.sync_copy(idx_hbm.at[pl.ds(wid * n_chunks, n_chunks)], idx_v)

        def body(i, carry):
            pltpu.async_copy(table_hbm.at[idx_v.at[i]], rows_v, sem).wait()
            pltpu.sync_copy(rows_v, out_hbm.at[pl.ds(base + i * _CHUNK, _CHUNK)])
            return carry

        lax.fori_loop(0, n_chunks, body, 0)

    return lookup


def kernel(inputs, word_embeddings):
    B, S = inputs.shape
    V, D = word_embeddings.shape
    N = B * S
    idx = inputs.reshape(N // _CHUNK, _CHUNK).astype(jnp.int32)
    out = _make_lookup(V, D, N)(idx, word_embeddings)
    return out.reshape(B, S, D)


# speedup vs baseline: 1.0914x; 1.0677x over previous
import functools

import jax
import jax.numpy as jnp
from jax import lax
from jax.experimental import pallas as pl
from jax.experimental.pallas import tpu as pltpu
from jax.experimental.pallas import tpu_sc as plsc

_NC = 2
_NS = 16
_NW = _NC * _NS

_CHUNK = 128
_NBUF = 4
_LOOK = 2


@functools.lru_cache(maxsize=None)
def _make_lookup(V, D, N):
    assert N % (_NW * _CHUNK) == 0
    b_per_w = N // _NW
    n_chunks = b_per_w // _CHUNK
    assert n_chunks % _NBUF == 0 and n_chunks > _NBUF

    mesh = plsc.VectorSubcoreMesh(
        core_axis_name="c", subcore_axis_name="s",
        num_cores=_NC, num_subcores=_NS)

    @functools.partial(
        pl.kernel,
        out_type=jax.ShapeDtypeStruct((N, D), jnp.float32),
        mesh=mesh,
        scratch_types=[
            pltpu.VMEM((n_chunks, _CHUNK), jnp.int32),
            pltpu.VMEM((_NBUF, _CHUNK, D), jnp.float32),
        ] + [pltpu.SemaphoreType.DMA] * (2 * _NBUF),
        compiler_params=pltpu.CompilerParams(use_tc_tiling_on_sc=False),
    )
    def lookup(idx_hbm, table_hbm, out_hbm, idx_v, rows, *sems):
        gsem = sems[:_NBUF]
        osem = sems[_NBUF:]
        wid = lax.axis_index("s") * _NC + lax.axis_index("c")
        base = wid * b_per_w

        pltpu.sync_copy(idx_hbm.at[pl.ds(wid * n_chunks, n_chunks)], idx_v)

        def start_gather(i, b):
            pltpu.async_copy(table_hbm.at[idx_v.at[i]], rows.at[b], gsem[b])

        def wait_gather(i, b):
            pltpu.make_async_copy(
                table_hbm.at[idx_v.at[i]], rows.at[b], gsem[b]).wait()

        def start_out(i, b):
            pltpu.async_copy(
                rows.at[b], out_hbm.at[pl.ds(base + i * _CHUNK, _CHUNK)],
                osem[b])

        def wait_out(b):
            pltpu.make_async_copy(
                rows.at[b], out_hbm.at[pl.ds(base, _CHUNK)], osem[b]).wait()

        for b in range(_LOOK):
            start_gather(b, b)

        def group(g, carry):
            for b in range(_NBUF):
                i = g * _NBUF + b
                wait_gather(i, b)
                start_out(i, b)
                j = i + _LOOK
                bj = (b + _LOOK) % _NBUF

                @pl.when(j < n_chunks)
                def _():
                    @pl.when(i >= _NBUF - _LOOK)
                    def _():
                        wait_out(bj)
                    start_gather(j, bj)
            return carry

        lax.fori_loop(0, n_chunks // _NBUF, group, 0)

        for b in range(_NBUF):
            wait_out(b)

    return lookup


def kernel(inputs, word_embeddings):
    B, S = inputs.shape
    V, D = word_embeddings.shape
    N = B * S
    idx = inputs.reshape(N // _CHUNK, _CHUNK).astype(jnp.int32)
    out = _make_lookup(V, D, N)(idx, word_embeddings)
    return out.reshape(B, S, D)


# speedup vs baseline: 1.0989x; 1.0068x over previous
import functools

import jax
import jax.numpy as jnp
from jax import lax
from jax.experimental import pallas as pl
from jax.experimental.pallas import tpu as pltpu
from jax.experimental.pallas import tpu_sc as plsc

_NC = 2
_NS = 16
_NW = _NC * _NS

_CHUNK = 512
_NBUF = 2
_LOOK = 1


@functools.lru_cache(maxsize=None)
def _make_lookup(V, D, N):
    assert N % (_NW * _CHUNK) == 0
    b_per_w = N // _NW
    n_chunks = b_per_w // _CHUNK
    assert n_chunks % _NBUF == 0 and n_chunks > _NBUF

    mesh = plsc.VectorSubcoreMesh(
        core_axis_name="c", subcore_axis_name="s",
        num_cores=_NC, num_subcores=_NS)

    @functools.partial(
        pl.kernel,
        out_type=jax.ShapeDtypeStruct((N, D), jnp.float32),
        mesh=mesh,
        scratch_types=[
            pltpu.VMEM((n_chunks, _CHUNK), jnp.int32),
            pltpu.VMEM((_NBUF, _CHUNK, D), jnp.float32),
        ] + [pltpu.SemaphoreType.DMA] * (2 * _NBUF),
        compiler_params=pltpu.CompilerParams(use_tc_tiling_on_sc=False),
    )
    def lookup(idx_hbm, table_hbm, out_hbm, idx_v, rows, *sems):
        gsem = sems[:_NBUF]
        osem = sems[_NBUF:]
        wid = lax.axis_index("s") * _NC + lax.axis_index("c")
        base = wid * b_per_w

        pltpu.sync_copy(idx_hbm.at[pl.ds(wid * n_chunks, n_chunks)], idx_v)

        def start_gather(i, b):
            pltpu.async_copy(table_hbm.at[idx_v.at[i]], rows.at[b], gsem[b])

        def wait_gather(i, b):
            pltpu.make_async_copy(
                table_hbm.at[idx_v.at[i]], rows.at[b], gsem[b]).wait()

        def start_out(i, b):
            pltpu.async_copy(
                rows.at[b], out_hbm.at[pl.ds(base + i * _CHUNK, _CHUNK)],
                osem[b])

        def wait_out(b):
            pltpu.make_async_copy(
                rows.at[b], out_hbm.at[pl.ds(base, _CHUNK)], osem[b]).wait()

        for b in range(_LOOK):
            start_gather(b, b)

        def group(g, carry):
            for b in range(_NBUF):
                i = g * _NBUF + b
                wait_gather(i, b)
                start_out(i, b)
                j = i + _LOOK
                bj = (b + _LOOK) % _NBUF

                @pl.when(j < n_chunks)
                def _():
                    @pl.when(i >= _NBUF - _LOOK)
                    def _():
                        wait_out(bj)
                    start_gather(j, bj)
            return carry

        lax.fori_loop(0, n_chunks // _NBUF, group, 0)

        for b in range(_NBUF):
            wait_out(b)

    return lookup


def kernel(inputs, word_embeddings):
    B, S = inputs.shape
    V, D = word_embeddings.shape
    N = B * S
    idx = inputs.reshape(N // _CHUNK, _CHUNK).astype(jnp.int32)
    out = _make_lookup(V, D, N)(idx, word_embeddings)
    return out.reshape(B, S, D)


# speedup vs baseline: 1.8425x; 1.6768x over previous
import functools

import jax
import jax.numpy as jnp
from jax import lax
from jax.experimental import pallas as pl
from jax.experimental.pallas import tpu as pltpu
from jax.experimental.pallas import tpu_sc as plsc

_NC = 2
_NS = 16
_NW = _NC * _NS


@functools.lru_cache(maxsize=None)
def _make_lookup(V, D, B, S):
    assert D % 8 == 0 and B % 128 == 0
    DT = D // 8
    NB = B // 128
    n_units = S * NB
    assert n_units % (2 * _NW) == 0
    u_per_w = n_units // _NW

    mesh = plsc.VectorSubcoreMesh(
        core_axis_name="c", subcore_axis_name="s",
        num_cores=_NC, num_subcores=_NS)

    @functools.partial(
        pl.kernel,
        out_type=jax.ShapeDtypeStruct((S, DT, NB, 8, 128), jnp.float32),
        mesh=mesh,
        scratch_types=[
            pltpu.VMEM((u_per_w, 128), jnp.int32),
            pltpu.VMEM((2, 128, D), jnp.float32),
            pltpu.VMEM((2, DT, 8, 128), jnp.float32),
        ] + [pltpu.SemaphoreType.DMA] * 4,
        compiler_params=pltpu.CompilerParams(
            use_tc_tiling_on_sc=False, needs_layout_passes=False),
    )
    def lookup(idx_hbm, table_hbm, out_hbm, idx_v, rows, tiles, g0, g1, o0, o1):
        gsem = (g0, g1)
        osem = (o0, o1)
        wid = lax.axis_index("s") * _NC + lax.axis_index("c")
        u0 = wid * u_per_w

        pltpu.sync_copy(idx_hbm.at[pl.ds(u0, u_per_w)], idx_v)

        iota = lax.iota(jnp.int32, 16)
        r_vec = lax.rem(iota, 8)
        tr_lo = lax.div(iota, 8)
        tr_hi = tr_lo + 2

        def start_gather(k, b):
            pltpu.async_copy(table_hbm.at[idx_v.at[k]], rows.at[b], gsem[b])

        def wait_gather(k, b):
            pltpu.make_async_copy(
                table_hbm.at[idx_v.at[k]], rows.at[b], gsem[b]).wait()

        def start_out(k, b):
            u = u0 + k
            s = u // NB
            tc = u % NB
            pltpu.async_copy(tiles.at[b], out_hbm.at[s, :, tc], osem[b])

        def wait_out(b):
            pltpu.make_async_copy(
                tiles.at[b], out_hbm.at[0, :, 0], osem[b]).wait()

        def transpose_unit(b):
            rb = rows.at[b]
            tb = tiles.at[b]

            def cblk(cb, carry):
                for ci in range(16):
                    c = cb * 16 + ci
                    c_splat = jnp.full((16,), 0, jnp.int32) + c
                    plsc.store_scatter(
                        tb, [tr_lo, r_vec, c_splat], rb[c, pl.ds(0, 16)])
                    plsc.store_scatter(
                        tb, [tr_hi, r_vec, c_splat], rb[c, pl.ds(16, 16)])
                return carry

            lax.fori_loop(0, 8, cblk, 0)

        start_gather(0, 0)

        def group(g, carry):
            for h in range(2):
                k = g * 2 + h
                b = h

                @pl.when(k + 1 < u_per_w)
                def _():
                    start_gather(k + 1, 1 - b)

                wait_gather(k, b)

                @pl.when(k >= 2)
                def _():
                    wait_out(b)

                transpose_unit(b)
                start_out(k, b)
            return carry

        lax.fori_loop(0, u_per_w // 2, group, 0)
        wait_out(0)
        wait_out(1)

    return lookup


def kernel(inputs, word_embeddings):
    B, S = inputs.shape
    V, D = word_embeddings.shape
    idx2 = inputs.T.reshape(S * (B // 128), 128).astype(jnp.int32)
    res = _make_lookup(V, D, B, S)(idx2, word_embeddings)
    return res.transpose(2, 4, 0, 1, 3).reshape(B, S, D)


# speedup vs baseline: 2.6059x; 1.4143x over previous
import functools

import jax
import jax.numpy as jnp
from jax import lax
from jax.experimental import pallas as pl
from jax.experimental.pallas import tpu as pltpu
from jax.experimental.pallas import tpu_sc as plsc

_NC = 2
_NS = 16
_NW = _NC * _NS


@functools.lru_cache(maxsize=None)
def _make_lookup(V, D, B, S):
    assert D % 8 == 0 and B % 128 == 0
    DT = D // 8
    NB = B // 128
    TCW = NB // _NW
    n_units = S * TCW
    assert NB % _NW == 0 and n_units % 2 == 0
    b_per_w = 128 * TCW

    mesh = plsc.VectorSubcoreMesh(
        core_axis_name="c", subcore_axis_name="s",
        num_cores=_NC, num_subcores=_NS)

    @functools.partial(
        pl.kernel,
        out_type=jax.ShapeDtypeStruct((S, DT, NB, 8, 128), jnp.float32),
        mesh=mesh,
        scratch_types=[
            pltpu.VMEM((b_per_w, S), jnp.int32),
            pltpu.VMEM((2, 128), jnp.int32),
            pltpu.VMEM((2, 128, D), jnp.float32),
            pltpu.VMEM((2, DT, 8, 129), jnp.float32),
        ] + [pltpu.SemaphoreType.DMA] * 4,
        compiler_params=pltpu.CompilerParams(
            use_tc_tiling_on_sc=False, needs_layout_passes=False),
    )
    def lookup(idx_hbm, table_hbm, out_hbm,
               idx_v, idx_c, rows, tiles, g0, g1, o0, o1):
        gsem = (g0, g1)
        osem = (o0, o1)
        wid = lax.axis_index("s") * _NC + lax.axis_index("c")

        pltpu.sync_copy(idx_hbm.at[pl.ds(wid * b_per_w, b_per_w)], idx_v)

        iota = lax.iota(jnp.int32, 16)
        r_vec = lax.rem(iota, 8)
        tr_lo = lax.div(iota, 8)
        tr_hi = tr_lo + 2

        def build_idx(k, b):
            j = k // S
            s = k % S
            s_splat = jnp.full((16,), 0, jnp.int32) + s
            for c0 in range(0, 128, 16):
                row_vec = iota + (128 * j + c0)
                vals = plsc.load_gather(idx_v, [row_vec, s_splat])
                idx_c[b, pl.ds(c0, 16)] = vals

        def start_gather(k, b):
            pltpu.async_copy(table_hbm.at[idx_c.at[b]], rows.at[b], gsem[b])

        def wait_gather(k, b):
            pltpu.make_async_copy(
                table_hbm.at[idx_c.at[b]], rows.at[b], gsem[b]).wait()

        def start_out(k, b):
            j = k // S
            s = k % S
            pltpu.async_copy(
                tiles.at[b, :, :, pl.ds(0, 128)],
                out_hbm.at[s, :, wid * TCW + j], osem[b])

        def wait_out(b):
            pltpu.make_async_copy(
                tiles.at[b, :, :, pl.ds(0, 128)],
                out_hbm.at[0, :, 0], osem[b]).wait()

        def transpose_unit(b):
            rb = rows.at[b]
            tb = tiles.at[b]

            def cblk(cb, carry):
                for ci in range(16):
                    c = cb * 16 + ci
                    c_splat = jnp.full((16,), 0, jnp.int32) + c
                    plsc.store_scatter(
                        tb, [tr_lo, r_vec, c_splat], rb[c, pl.ds(0, 16)])
                    plsc.store_scatter(
                        tb, [tr_hi, r_vec, c_splat], rb[c, pl.ds(16, 16)])
                return carry

            lax.fori_loop(0, 8, cblk, 0)

        build_idx(0, 0)
        start_gather(0, 0)

        def group(g, carry):
            for h in range(2):
                k = g * 2 + h
                b = h

                @pl.when(k + 1 < n_units)
                def _():
                    build_idx(k + 1, 1 - b)
                    start_gather(k + 1, 1 - b)

                wait_gather(k, b)

                @pl.when(k >= 2)
                def _():
                    wait_out(b)

                transpose_unit(b)
                start_out(k, b)
            return carry

        lax.fori_loop(0, n_units // 2, group, 0)
        wait_out(0)
        wait_out(1)

    return lookup


def kernel(inputs, word_embeddings):
    B, S = inputs.shape
    V, D = word_embeddings.shape
    res = _make_lookup(V, D, B, S)(inputs, word_embeddings)
    return res.transpose(2, 4, 0, 1, 3).reshape(B, S, D)
